# Initial kernel scaffold; baseline (speedup 1.0000x reference)
import jax
import jax.numpy as jnp
from jax.experimental import pallas as pl


def kernel(x, type_ids, W, b):
    raise NotImplementedError("write your pallas kernel here")


# speedup vs baseline: 1.6772x; 1.6772x over previous
import jax
import jax.numpy as jnp
from jax.experimental import pallas as pl

N = 4096
D = 1024
OUT = 512
E = 8

BN = 1024
NB = N // BN


def _dense_masked_body(tid_ref, x_ref, w_ref, b_ref, out_ref):
    e = pl.program_id(1)

    @pl.when(e == 0)
    def _init():
        out_ref[...] = jnp.zeros_like(out_ref)

    tid = tid_ref[0]
    mask = tid == e
    acc = jnp.dot(x_ref[...], w_ref[0], preferred_element_type=jnp.float32)
    acc = acc + b_ref[0, 0][None, :]
    out_ref[...] += jnp.where(mask, acc, 0.0)


def kernel(x, type_ids, W, b):
    tid = type_ids.astype(jnp.int32)
    tid3 = tid.reshape(NB, BN, 1)
    b3 = b.reshape(E, 1, OUT)

    feats = pl.pallas_call(
        _dense_masked_body,
        grid=(NB, E),
        in_specs=[
            pl.BlockSpec((1, BN, 1), lambda i, e: (i, 0, 0)),
            pl.BlockSpec((BN, D), lambda i, e: (i, 0)),
            pl.BlockSpec((1, D, OUT), lambda i, e: (e, 0, 0)),
            pl.BlockSpec((1, 1, OUT), lambda i, e: (e, 0, 0)),
        ],
        out_specs=pl.BlockSpec((BN, OUT), lambda i, e: (i, 0)),
        out_shape=jax.ShapeDtypeStruct((N, OUT), jnp.float32),
    )(tid3, x, W, b3)

    onehot = (tid[:, None] == jnp.arange(E, dtype=jnp.int32)[None, :]).astype(
        jnp.float32)
    out = jnp.concatenate([feats, onehot], axis=-1)
    return out[:, None, :]
